# Initial kernel scaffold; baseline (speedup 1.0000x reference)
import jax
import jax.numpy as jnp
from jax.experimental import pallas as pl


def kernel(type_indices, positions, relation_indices, edge_attrs, node_type_table, edge_type_table, pos_W, pos_b, attr_W, attr_b):
    raise NotImplementedError("write your pallas kernel here")


# speedup vs baseline: 1.9709x; 1.9709x over previous
import jax
import jax.numpy as jnp
from jax import lax
from jax.experimental import pallas as pl
from jax.experimental.pallas import tpu as pltpu, tpu_sc as plsc

N_NODES = 100000
N_EDGES = 3200000
NODE_D = 128
EDGE_D = 16

_INFO = plsc.get_sparse_core_info()
NC = _INFO.num_cores
NS = _INFO.num_subcores
NW = NC * NS

NPT = 3128
EPT = N_EDGES // NW

CN = 512
CE = 2000

_I32 = jnp.int32


def _splat(v, n=16):
    return jnp.full((n,), v, _I32)


def _body(tidx_hbm, pos_hbm, rel_hbm, attrs_hbm, ntab_hbm, nwf_hbm,
          etabT_hbm, attrW_hbm, node_out, edge_out,
          ntab_v, nwf_v, etabT_v, attrW_v,
          tidx_v, pos_v, nout_v, rel_v, attrs_v, eout_v):
    wid = lax.axis_index("s") * NC + lax.axis_index("c")

    pltpu.sync_copy(ntab_hbm, ntab_v)
    pltpu.sync_copy(nwf_hbm, nwf_v)
    pltpu.sync_copy(etabT_hbm, etabT_v)
    pltpu.sync_copy(attrW_hbm, attrW_v)

    iota = lax.iota(_I32, 16)

    nw0 = [nwf_v[pl.ds(64 + j * 16, 16)] for j in range(4)]
    nw1 = [nwf_v[pl.ds(192 + j * 16, 16)] for j in range(4)]

    ebase = wid * EPT

    def edge_chunk(c, carry):
        b = ebase + c * CE
        pltpu.sync_copy(rel_hbm.at[pl.ds(b, CE)], rel_v.at[pl.ds(0, CE)])
        pltpu.sync_copy(attrs_hbm.at[pl.ds(2 * b, 2 * CE)],
                        attrs_v.at[pl.ds(0, 2 * CE)])

        def egroup(g, carry2):
            ew0 = [plsc.load_gather(attrW_v, [_splat(8 + k)]) for k in range(8)]
            ew1 = [plsc.load_gather(attrW_v, [_splat(16 + k)]) for k in range(8)]
            rows = _splat(0) + (g * 16) + iota
            rel16 = rel_v[pl.ds(g * 16, 16)]
            a0 = plsc.load_gather(attrs_v, [rows * 2])
            a1 = plsc.load_gather(attrs_v, [rows * 2 + 1])
            out_base = rows * EDGE_D
            for col in range(EDGE_D):
                v = plsc.load_gather(etabT_v, [_splat(col * 5) + rel16])
                if col >= 8:
                    v = v + a0 * ew0[col - 8] + a1 * ew1[col - 8]
                plsc.store_scatter(eout_v, [out_base + col], v)
            return carry2

        lax.fori_loop(0, CE // 16, egroup, 0)
        pltpu.sync_copy(eout_v, edge_out.at[pl.ds(b * EDGE_D, CE * EDGE_D)])
        return carry

    lax.fori_loop(0, EPT // CE, edge_chunk, 0)

    nbase = wid * NPT
    ncount = jnp.minimum(NPT, N_NODES - nbase)

    def node_at(b):
        pltpu.sync_copy(tidx_hbm.at[pl.ds(b, CN)], tidx_v)
        pltpu.sync_copy(pos_hbm.at[pl.ds(2 * b, 2 * CN)], pos_v)

        def nrow(n, carry2):
            nsp = _splat(0) + n
            t = plsc.load_gather(tidx_v, [nsp])
            p0 = plsc.load_gather(pos_v, [nsp * 2])
            p1 = plsc.load_gather(pos_v, [nsp * 2 + 1])
            rowbase = t * NODE_D
            for j in range(NODE_D // 16):
                v = plsc.load_gather(ntab_v, [rowbase + (j * 16) + iota])
                if j >= 4:
                    v = v + p0 * nw0[j - 4] + p1 * nw1[j - 4]
                nout_v[pl.ds(n * NODE_D + j * 16, 16)] = v
            return carry2

        lax.fori_loop(0, CN, nrow, 0)
        pltpu.sync_copy(nout_v, node_out.at[pl.ds(b * NODE_D, CN * NODE_D)])

    n_full = ncount // CN

    def node_chunk(c, carry):
        node_at(nbase + c * CN)
        return carry

    lax.fori_loop(0, n_full, node_chunk, 0)

    rem = ncount - n_full * CN

    @pl.when(rem > 0)
    def _():
        node_at(nbase + ncount - CN)


def kernel(type_indices, positions, relation_indices, edge_attrs,
           node_type_table, edge_type_table, pos_W, pos_b, attr_W, attr_b):
    ntab = jnp.concatenate(
        [node_type_table, jnp.broadcast_to(pos_b, (5, NODE_D // 2))],
        axis=1).reshape(-1)
    nwf = jnp.concatenate(
        [jnp.zeros((2, NODE_D // 2), jnp.float32), pos_W], axis=1).reshape(-1)
    etabT = jnp.concatenate(
        [edge_type_table.T,
         jnp.broadcast_to(attr_b[:, None], (EDGE_D // 2, 5))],
        axis=0).reshape(-1)
    etabT = jnp.pad(etabT, (0, 128 - etabT.shape[0]))
    attrW_pad = jnp.pad(attr_W.reshape(-1), (8, 120 - 2 * (EDGE_D // 2)))

    mesh = plsc.VectorSubcoreMesh(core_axis_name="c", subcore_axis_name="s")
    fn = pl.kernel(
        _body,
        mesh=mesh,
        compiler_params=pltpu.CompilerParams(needs_layout_passes=False),
        out_type=[
            jax.ShapeDtypeStruct((N_NODES * NODE_D,), jnp.float32),
            jax.ShapeDtypeStruct((N_EDGES * EDGE_D,), jnp.float32),
        ],
        scratch_types=[
            pltpu.VMEM((5 * NODE_D,), jnp.float32),
            pltpu.VMEM((2 * NODE_D,), jnp.float32),
            pltpu.VMEM((128,), jnp.float32),
            pltpu.VMEM((128,), jnp.float32),
            pltpu.VMEM((CN,), jnp.int32),
            pltpu.VMEM((2 * CN,), jnp.float32),
            pltpu.VMEM((CN * NODE_D,), jnp.float32),
            pltpu.VMEM((2048,), jnp.int32),
            pltpu.VMEM((4096,), jnp.float32),
            pltpu.VMEM((CE * EDGE_D,), jnp.float32),
        ],
    )
    node_flat, edge_flat = fn(
        type_indices, positions.reshape(-1), relation_indices,
        edge_attrs.reshape(-1), ntab, nwf, etabT, attrW_pad)
    return (node_flat.reshape(N_NODES, NODE_D),
            edge_flat.reshape(N_EDGES, EDGE_D))


# speedup vs baseline: 2.0616x; 1.0460x over previous
import jax
import jax.numpy as jnp
from jax import lax
from jax.experimental import pallas as pl
from jax.experimental.pallas import tpu as pltpu, tpu_sc as plsc

N_NODES = 100000
N_EDGES = 3200000
NODE_D = 128
EDGE_D = 16

_INFO = plsc.get_sparse_core_info()
NC = _INFO.num_cores
NS = _INFO.num_subcores
NW = NC * NS

NPT = 3128
EPT = N_EDGES // NW

CN = 512
CE = 2000

_I32 = jnp.int32


def _splat(v, n=16):
    return jnp.full((n,), v, _I32)


def _body(tidx_hbm, pos_hbm, rel_hbm, attrs_hbm, ntab_hbm, nwf_hbm,
          etabT_hbm, attrW_hbm, node_out, edge_out,
          ntab_v, nwf_v, etabT_v, attrW_v,
          tidx_v, pos_v, nout_v, rel_v, attrs_v, eout_v):
    wid = lax.axis_index("s") * NC + lax.axis_index("c")

    pltpu.sync_copy(ntab_hbm, ntab_v)
    pltpu.sync_copy(nwf_hbm, nwf_v)
    pltpu.sync_copy(etabT_hbm, etabT_v)
    pltpu.sync_copy(attrW_hbm, attrW_v)

    iota = lax.iota(_I32, 16)

    nw0 = [nwf_v[pl.ds(64 + j * 16, 16)] for j in range(4)]
    nw1 = [nwf_v[pl.ds(192 + j * 16, 16)] for j in range(4)]

    ebase = wid * EPT

    def edge_chunk(c, carry):
        b = ebase + c * CE
        pltpu.sync_copy(rel_hbm.at[pl.ds(b, CE)], rel_v.at[pl.ds(0, CE)])
        pltpu.sync_copy(attrs_hbm.at[pl.ds(2 * b, 2 * CE)],
                        attrs_v.at[pl.ds(0, 2 * CE)])

        @plsc.parallel_loop(0, CE // 16, unroll=4)
        def egroup(g):
            ew0 = [plsc.load_gather(attrW_v, [_splat(8 + k)]) for k in range(8)]
            ew1 = [plsc.load_gather(attrW_v, [_splat(16 + k)]) for k in range(8)]
            rows = _splat(0) + (g * 16) + iota
            rel16 = rel_v[pl.ds(g * 16, 16)]
            a0 = plsc.load_gather(attrs_v, [rows * 2])
            a1 = plsc.load_gather(attrs_v, [rows * 2 + 1])
            out_base = rows * EDGE_D
            for col in range(EDGE_D):
                v = plsc.load_gather(etabT_v, [_splat(col * 5) + rel16])
                if col >= 8:
                    v = v + a0 * ew0[col - 8] + a1 * ew1[col - 8]
                plsc.store_scatter(eout_v, [out_base + col], v)
        pltpu.sync_copy(eout_v, edge_out.at[pl.ds(b * EDGE_D, CE * EDGE_D)])
        return carry

    lax.fori_loop(0, EPT // CE, edge_chunk, 0)

    nbase = wid * NPT
    ncount = jnp.minimum(NPT, N_NODES - nbase)

    def node_at(b):
        pltpu.sync_copy(tidx_hbm.at[pl.ds(b, CN)], tidx_v)
        pltpu.sync_copy(pos_hbm.at[pl.ds(2 * b, 2 * CN)], pos_v)

        @plsc.parallel_loop(0, CN, unroll=4)
        def nrow(n):
            nsp = _splat(0) + n
            t = plsc.load_gather(tidx_v, [nsp])
            p0 = plsc.load_gather(pos_v, [nsp * 2])
            p1 = plsc.load_gather(pos_v, [nsp * 2 + 1])
            rowbase = t * NODE_D
            for j in range(NODE_D // 16):
                v = plsc.load_gather(ntab_v, [rowbase + (j * 16) + iota])
                if j >= 4:
                    v = v + p0 * nw0[j - 4] + p1 * nw1[j - 4]
                nout_v[pl.ds(n * NODE_D + j * 16, 16)] = v
        pltpu.sync_copy(nout_v, node_out.at[pl.ds(b * NODE_D, CN * NODE_D)])

    n_full = ncount // CN

    def node_chunk(c, carry):
        node_at(nbase + c * CN)
        return carry

    lax.fori_loop(0, n_full, node_chunk, 0)

    rem = ncount - n_full * CN

    @pl.when(rem > 0)
    def _():
        node_at(nbase + ncount - CN)


def kernel(type_indices, positions, relation_indices, edge_attrs,
           node_type_table, edge_type_table, pos_W, pos_b, attr_W, attr_b):
    ntab = jnp.concatenate(
        [node_type_table, jnp.broadcast_to(pos_b, (5, NODE_D // 2))],
        axis=1).reshape(-1)
    nwf = jnp.concatenate(
        [jnp.zeros((2, NODE_D // 2), jnp.float32), pos_W], axis=1).reshape(-1)
    etabT = jnp.concatenate(
        [edge_type_table.T,
         jnp.broadcast_to(attr_b[:, None], (EDGE_D // 2, 5))],
        axis=0).reshape(-1)
    etabT = jnp.pad(etabT, (0, 128 - etabT.shape[0]))
    attrW_pad = jnp.pad(attr_W.reshape(-1), (8, 120 - 2 * (EDGE_D // 2)))

    mesh = plsc.VectorSubcoreMesh(core_axis_name="c", subcore_axis_name="s")
    fn = pl.kernel(
        _body,
        mesh=mesh,
        compiler_params=pltpu.CompilerParams(needs_layout_passes=False),
        out_type=[
            jax.ShapeDtypeStruct((N_NODES * NODE_D,), jnp.float32),
            jax.ShapeDtypeStruct((N_EDGES * EDGE_D,), jnp.float32),
        ],
        scratch_types=[
            pltpu.VMEM((5 * NODE_D,), jnp.float32),
            pltpu.VMEM((2 * NODE_D,), jnp.float32),
            pltpu.VMEM((128,), jnp.float32),
            pltpu.VMEM((128,), jnp.float32),
            pltpu.VMEM((CN,), jnp.int32),
            pltpu.VMEM((2 * CN,), jnp.float32),
            pltpu.VMEM((CN * NODE_D,), jnp.float32),
            pltpu.VMEM((2048,), jnp.int32),
            pltpu.VMEM((4096,), jnp.float32),
            pltpu.VMEM((CE * EDGE_D,), jnp.float32),
        ],
    )
    node_flat, edge_flat = fn(
        type_indices, positions.reshape(-1), relation_indices,
        edge_attrs.reshape(-1), ntab, nwf, etabT, attrW_pad)
    return (node_flat.reshape(N_NODES, NODE_D),
            edge_flat.reshape(N_EDGES, EDGE_D))


# speedup vs baseline: 28.8296x; 13.9840x over previous
import jax
import jax.numpy as jnp
from jax import lax
from jax.experimental import pallas as pl
from jax.experimental.pallas import tpu as pltpu, tpu_sc as plsc

N_NODES = 100000
N_EDGES = 3200000
NODE_D = 128
EDGE_D = 16
NBLK = N_EDGES // 128

_INFO = plsc.get_sparse_core_info()
NC = _INFO.num_cores
NS = _INFO.num_subcores
NW = NC * NS

NPT = 3128
BPT = 784

CN = 256
CEB = 8
CE = CEB * 128

_I32 = jnp.int32


def _splat(v, n=16):
    return jnp.full((n,), v, _I32)


def _body(tidx_hbm, p0_hbm, p1_hbm, rel_hbm, af_hbm, ntab_hbm, nwf_hbm,
          etabT_hbm, attrW_hbm, node_out, edge_out,
          ntab_v, nwf_v, etabT_v, attrW_v,
          tidx_v, p0_v, p1_v, nout_v, rel_v, attrs_v, eout_v):
    wid = lax.axis_index("s") * NC + lax.axis_index("c")

    pltpu.sync_copy(ntab_hbm, ntab_v)
    pltpu.sync_copy(nwf_hbm, nwf_v)
    pltpu.sync_copy(etabT_hbm, etabT_v)
    pltpu.sync_copy(attrW_hbm, attrW_v)

    iota = lax.iota(_I32, 16)

    nw0 = [nwf_v[pl.ds(64 + j * 16, 16)] for j in range(4)]
    nw1 = [nwf_v[pl.ds(192 + j * 16, 16)] for j in range(4)]

    tb_base = wid * BPT
    bcount = jnp.minimum(BPT, NBLK - tb_base)

    def edge_chunk(c, carry):
        tb = tb_base + c * CEB
        pltpu.sync_copy(rel_hbm.at[pl.ds(tb * 128, CE)], rel_v)
        pltpu.sync_copy(af_hbm.at[pl.ds(tb * 256, 2 * CE)], attrs_v)

        @plsc.parallel_loop(0, CE // 16, unroll=8)
        def egroup(g):
            ew0 = [plsc.load_gather(attrW_v, [_splat(8 + k)]) for k in range(8)]
            ew1 = [plsc.load_gather(attrW_v, [_splat(16 + k)]) for k in range(8)]
            teb = g // 8
            off = (g % 8) * 16
            rel16 = rel_v[pl.ds(g * 16, 16)]
            a0 = attrs_v[pl.ds(teb * 256 + off, 16)]
            a1 = attrs_v[pl.ds(teb * 256 + 128 + off, 16)]
            for col in range(EDGE_D):
                v = plsc.load_gather(etabT_v, [_splat(col * 5) + rel16])
                if col >= 8:
                    v = v + a0 * ew0[col - 8] + a1 * ew1[col - 8]
                eout_v[col // 8, teb, col % 8, pl.ds(off, 16)] = v

        pltpu.sync_copy(eout_v.at[0], edge_out.at[0, pl.ds(tb, CEB)])
        pltpu.sync_copy(eout_v.at[1], edge_out.at[1, pl.ds(tb, CEB)])
        return carry

    lax.fori_loop(0, bcount // CEB, edge_chunk, 0)

    nbase = wid * NPT
    ncount = jnp.minimum(NPT, N_NODES - nbase)

    def node_at(b):
        pltpu.sync_copy(tidx_hbm.at[pl.ds(b, CN)], tidx_v)
        pltpu.sync_copy(p0_hbm.at[pl.ds(b, CN)], p0_v)
        pltpu.sync_copy(p1_hbm.at[pl.ds(b, CN)], p1_v)

        @plsc.parallel_loop(0, CN, unroll=4)
        def nrow(n):
            nsp = _splat(0) + n
            t = plsc.load_gather(tidx_v, [nsp])
            q0 = plsc.load_gather(p0_v, [nsp])
            q1 = plsc.load_gather(p1_v, [nsp])
            rowbase = t * NODE_D
            for j in range(NODE_D // 16):
                v = plsc.load_gather(ntab_v, [rowbase + (j * 16) + iota])
                if j >= 4:
                    v = v + q0 * nw0[j - 4] + q1 * nw1[j - 4]
                nout_v[n // 8, n % 8, pl.ds(j * 16, 16)] = v

        pltpu.sync_copy(nout_v, node_out.at[pl.ds(b // 8, CN // 8)])

    n_full = ncount // CN

    def node_chunk(c, carry):
        node_at(nbase + c * CN)
        return carry

    lax.fori_loop(0, n_full, node_chunk, 0)

    rem = ncount - n_full * CN

    @pl.when(rem > 0)
    def _():
        node_at(nbase + ncount - CN)


def kernel(type_indices, positions, relation_indices, edge_attrs,
           node_type_table, edge_type_table, pos_W, pos_b, attr_W, attr_b):
    ntab = jnp.concatenate(
        [node_type_table, jnp.broadcast_to(pos_b, (5, NODE_D // 2))],
        axis=1).reshape(-1)
    nwf = jnp.concatenate(
        [jnp.zeros((2, NODE_D // 2), jnp.float32), pos_W], axis=1).reshape(-1)
    etabT = jnp.concatenate(
        [edge_type_table.T,
         jnp.broadcast_to(attr_b[:, None], (EDGE_D // 2, 5))],
        axis=0).reshape(-1)
    etabT = jnp.pad(etabT, (0, 128 - etabT.shape[0]))
    attrW_pad = jnp.pad(attr_W.reshape(-1), (8, 120 - 2 * (EDGE_D // 2)))

    af = edge_attrs.reshape(NBLK, 128, 2).transpose(0, 2, 1).reshape(-1)
    p0 = positions[:, 0]
    p1 = positions[:, 1]

    mesh = plsc.VectorSubcoreMesh(core_axis_name="c", subcore_axis_name="s")
    fn = pl.kernel(
        _body,
        mesh=mesh,
        compiler_params=pltpu.CompilerParams(needs_layout_passes=False),
        out_type=[
            jax.ShapeDtypeStruct((N_NODES // 8, 8, NODE_D), jnp.float32),
            jax.ShapeDtypeStruct((2, NBLK, 8, 128), jnp.float32),
        ],
        scratch_types=[
            pltpu.VMEM((5 * NODE_D,), jnp.float32),
            pltpu.VMEM((2 * NODE_D,), jnp.float32),
            pltpu.VMEM((128,), jnp.float32),
            pltpu.VMEM((128,), jnp.float32),
            pltpu.VMEM((CN,), jnp.int32),
            pltpu.VMEM((CN,), jnp.float32),
            pltpu.VMEM((CN,), jnp.float32),
            pltpu.VMEM((CN // 8, 8, NODE_D), jnp.float32),
            pltpu.VMEM((CE,), jnp.int32),
            pltpu.VMEM((2 * CE,), jnp.float32),
            pltpu.VMEM((2, CEB, 8, 128), jnp.float32),
        ],
    )
    node3, edge4 = fn(
        type_indices, p0, p1, relation_indices, af,
        ntab, nwf, etabT, attrW_pad)
    node_features = node3.reshape(N_NODES, NODE_D)
    edge_attr = edge4.transpose(1, 3, 0, 2).reshape(N_EDGES, EDGE_D)
    return (node_features, edge_attr)


# speedup vs baseline: 52.7998x; 1.8314x over previous
import jax
import jax.numpy as jnp
from jax import lax
from jax.experimental import pallas as pl
from jax.experimental.pallas import tpu as pltpu, tpu_sc as plsc

N_NODES = 100000
N_EDGES = 3200000
NODE_D = 128
EDGE_D = 16
NBLK = N_EDGES // 128

_INFO = plsc.get_sparse_core_info()
NC = _INFO.num_cores
NS = _INFO.num_subcores
NW = NC * NS

NPT = 3128
BPT = 784

CN = 256
CEB = 8
CE = CEB * 128

_I32 = jnp.int32


def _splat(v, n=16):
    return jnp.full((n,), v, _I32)


def _body(tidx_hbm, p0_hbm, p1_hbm, rel_hbm, af_hbm, ntab_hbm, nwf_hbm,
          etabT_hbm, attrW_hbm, node_out, edge_out,
          ntab_v, nwf_v, etabT_v, attrW_v,
          tidx_v, p0_v, p1_v, nout_v, rel_v, attrs_v, eout_v,
          isem0, isem1, osem0, osem1):
    wid = lax.axis_index("s") * NC + lax.axis_index("c")
    isems = (isem0, isem1)
    osems = (osem0, osem1)

    pltpu.sync_copy(ntab_hbm, ntab_v)
    pltpu.sync_copy(nwf_hbm, nwf_v)
    pltpu.sync_copy(etabT_hbm, etabT_v)
    pltpu.sync_copy(attrW_hbm, attrW_v)

    iota = lax.iota(_I32, 16)

    nw0 = [nwf_v[pl.ds(64 + j * 16, 16)] for j in range(4)]
    nw1 = [nwf_v[pl.ds(192 + j * 16, 16)] for j in range(4)]
    nbias = [ntab_v[pl.ds(64 + j * 16, 16)] for j in range(4)]

    tb_base = wid * BPT
    bcount = jnp.minimum(BPT, NBLK - tb_base)
    nch = bcount // CEB

    def start_in(c, b):
        tb = tb_base + c * CEB
        pltpu.async_copy(rel_hbm.at[pl.ds(tb * 128, CE)], rel_v.at[b],
                         isems[b])
        pltpu.async_copy(af_hbm.at[pl.ds(tb * 256, 2 * CE)], attrs_v.at[b],
                         isems[b])

    def wait_in(b):
        pltpu.make_async_copy(rel_hbm.at[pl.ds(0, CE)], rel_v.at[b],
                              isems[b]).wait()
        pltpu.make_async_copy(af_hbm.at[pl.ds(0, 2 * CE)], attrs_v.at[b],
                              isems[b]).wait()

    def wait_out(b):
        pltpu.make_async_copy(eout_v.at[b, 0], edge_out.at[0, pl.ds(0, CEB)],
                              osems[b]).wait()
        pltpu.make_async_copy(eout_v.at[b, 1], edge_out.at[1, pl.ds(0, CEB)],
                              osems[b]).wait()

    def do_chunk(c, b):
        wait_in(b)

        @pl.when(c >= 2)
        def _():
            wait_out(b)

        @plsc.parallel_loop(0, CE // 16, unroll=8)
        def egroup(g):
            ew0 = [plsc.load_gather(attrW_v, [_splat(8 + k)])
                   for k in range(8)]
            ew1 = [plsc.load_gather(attrW_v, [_splat(16 + k)])
                   for k in range(8)]
            teb = g // 8
            off = (g % 8) * 16
            rel16 = rel_v[b, pl.ds(g * 16, 16)]
            a0 = attrs_v[b, pl.ds(teb * 256 + off, 16)]
            a1 = attrs_v[b, pl.ds(teb * 256 + 128 + off, 16)]
            for col in range(EDGE_D):
                v = plsc.load_gather(etabT_v, [_splat(col * 5) + rel16])
                if col >= 8:
                    v = v + a0 * ew0[col - 8] + a1 * ew1[col - 8]
                eout_v[b, col // 8, teb, col % 8, pl.ds(off, 16)] = v

        tb = tb_base + c * CEB
        pltpu.async_copy(eout_v.at[b, 0], edge_out.at[0, pl.ds(tb, CEB)],
                         osems[b])
        pltpu.async_copy(eout_v.at[b, 1], edge_out.at[1, pl.ds(tb, CEB)],
                         osems[b])
        start_in(jnp.minimum(c + 2, nch - 1), b)

    start_in(0, 0)
    start_in(jnp.minimum(1, nch - 1), 1)

    def chunk_body(c, carry):
        @pl.when(c % 2 == 0)
        def _():
            do_chunk(c, 0)

        @pl.when(c % 2 == 1)
        def _():
            do_chunk(c, 1)

        return carry

    lax.fori_loop(0, nch, chunk_body, 0)
    wait_in(0)
    wait_in(1)
    wait_out(0)
    wait_out(1)

    nbase = wid * NPT
    ncount = jnp.minimum(NPT, N_NODES - nbase)

    def node_at(nb):
        pltpu.sync_copy(tidx_hbm.at[pl.ds(nb, CN)], tidx_v)
        pltpu.sync_copy(p0_hbm.at[pl.ds(nb, CN)], p0_v)
        pltpu.sync_copy(p1_hbm.at[pl.ds(nb, CN)], p1_v)

        @plsc.parallel_loop(0, CN, unroll=4)
        def nrow(n):
            nsp = _splat(0) + n
            t = plsc.load_gather(tidx_v, [nsp])
            q0 = plsc.load_gather(p0_v, [nsp])
            q1 = plsc.load_gather(p1_v, [nsp])
            rowbase = t * NODE_D
            for j in range(4):
                v = plsc.load_gather(ntab_v, [rowbase + (j * 16) + iota])
                nout_v[n // 8, n % 8, pl.ds(j * 16, 16)] = v
            for j in range(4):
                v = nbias[j] + q0 * nw0[j] + q1 * nw1[j]
                nout_v[n // 8, n % 8, pl.ds(64 + j * 16, 16)] = v

        pltpu.sync_copy(nout_v, node_out.at[pl.ds(nb // 8, CN // 8)])

    n_full = ncount // CN

    def node_chunk(c, carry):
        node_at(nbase + c * CN)
        return carry

    lax.fori_loop(0, n_full, node_chunk, 0)

    rem = ncount - n_full * CN

    @pl.when(rem > 0)
    def _():
        node_at(nbase + ncount - CN)


def kernel(type_indices, positions, relation_indices, edge_attrs,
           node_type_table, edge_type_table, pos_W, pos_b, attr_W, attr_b):
    ntab = jnp.concatenate(
        [node_type_table, jnp.broadcast_to(pos_b, (5, NODE_D // 2))],
        axis=1).reshape(-1)
    nwf = jnp.concatenate(
        [jnp.zeros((2, NODE_D // 2), jnp.float32), pos_W], axis=1).reshape(-1)
    etabT = jnp.concatenate(
        [edge_type_table.T,
         jnp.broadcast_to(attr_b[:, None], (EDGE_D // 2, 5))],
        axis=0).reshape(-1)
    etabT = jnp.pad(etabT, (0, 128 - etabT.shape[0]))
    attrW_pad = jnp.pad(attr_W.reshape(-1), (8, 120 - 2 * (EDGE_D // 2)))

    af = edge_attrs.reshape(NBLK, 128, 2).transpose(0, 2, 1).reshape(-1)
    p0 = positions[:, 0]
    p1 = positions[:, 1]

    mesh = plsc.VectorSubcoreMesh(core_axis_name="c", subcore_axis_name="s")
    fn = pl.kernel(
        _body,
        mesh=mesh,
        compiler_params=pltpu.CompilerParams(needs_layout_passes=False),
        out_type=[
            jax.ShapeDtypeStruct((N_NODES // 8, 8, NODE_D), jnp.float32),
            jax.ShapeDtypeStruct((2, NBLK, 8, 128), jnp.float32),
        ],
        scratch_types=[
            pltpu.VMEM((5 * NODE_D,), jnp.float32),
            pltpu.VMEM((2 * NODE_D,), jnp.float32),
            pltpu.VMEM((128,), jnp.float32),
            pltpu.VMEM((128,), jnp.float32),
            pltpu.VMEM((CN,), jnp.int32),
            pltpu.VMEM((CN,), jnp.float32),
            pltpu.VMEM((CN,), jnp.float32),
            pltpu.VMEM((CN // 8, 8, NODE_D), jnp.float32),
            pltpu.VMEM((2, CE), jnp.int32),
            pltpu.VMEM((2, 2 * CE), jnp.float32),
            pltpu.VMEM((2, 2, CEB, 8, 128), jnp.float32),
            pltpu.SemaphoreType.DMA,
            pltpu.SemaphoreType.DMA,
            pltpu.SemaphoreType.DMA,
            pltpu.SemaphoreType.DMA,
        ],
    )
    node3, edge4 = fn(
        type_indices, p0, p1, relation_indices, af,
        ntab, nwf, etabT, attrW_pad)
    node_features = node3.reshape(N_NODES, NODE_D)
    edge_attr = edge4.transpose(1, 3, 0, 2).reshape(N_EDGES, EDGE_D)
    return (node_features, edge_attr)


# speedup vs baseline: 63.0793x; 1.1947x over previous
import jax
import jax.numpy as jnp
from jax import lax
from jax.experimental import pallas as pl
from jax.experimental.pallas import tpu as pltpu, tpu_sc as plsc

N_NODES = 100000
N_EDGES = 3200000
NODE_D = 128
EDGE_D = 16
NBLK = N_EDGES // 128

_INFO = plsc.get_sparse_core_info()
NC = _INFO.num_cores
NS = _INFO.num_subcores
NW = NC * NS

NPT = 3128
BPT = 784

CN = 256
CEB = 8
CE = CEB * 128

_I32 = jnp.int32


def _splat(v, n=16):
    return jnp.full((n,), v, _I32)


def _body(tidx_hbm, p0_hbm, p1_hbm, rel_hbm, af_hbm, ntab_hbm, nwf_hbm,
          etabT_hbm, attrW_hbm, node_out, edge_out,
          ntab_v, nwf_v, etabT_v, attrW_v,
          tidx_v, p0_v, p1_v, nout_v, rel_v, attrs_v, eout_v,
          isem0, isem1, osem0, osem1):
    wid = lax.axis_index("s") * NC + lax.axis_index("c")
    isems = (isem0, isem1)
    osems = (osem0, osem1)

    pltpu.sync_copy(ntab_hbm, ntab_v)
    pltpu.sync_copy(nwf_hbm, nwf_v)
    pltpu.sync_copy(etabT_hbm, etabT_v)
    pltpu.sync_copy(attrW_hbm, attrW_v)

    iota = lax.iota(_I32, 16)

    nw0 = [nwf_v[pl.ds(64 + j * 16, 16)] for j in range(4)]
    nw1 = [nwf_v[pl.ds(192 + j * 16, 16)] for j in range(4)]
    nbias = [ntab_v[pl.ds(64 + j * 16, 16)] for j in range(4)]

    tb_base = wid * BPT
    bcount = jnp.minimum(BPT, NBLK - tb_base)
    nch = bcount // CEB

    def start_in(c, b):
        tb = tb_base + c * CEB
        pltpu.async_copy(rel_hbm.at[pl.ds(tb * 128, CE)], rel_v.at[b],
                         isems[b])
        pltpu.async_copy(af_hbm.at[pl.ds(tb * 256, 2 * CE)], attrs_v.at[b],
                         isems[b])

    def wait_in(b):
        pltpu.make_async_copy(rel_hbm.at[pl.ds(0, CE)], rel_v.at[b],
                              isems[b]).wait()
        pltpu.make_async_copy(af_hbm.at[pl.ds(0, 2 * CE)], attrs_v.at[b],
                              isems[b]).wait()

    def wait_out(b):
        pltpu.make_async_copy(eout_v.at[b, 0], edge_out.at[0, pl.ds(0, CEB)],
                              osems[b]).wait()
        pltpu.make_async_copy(eout_v.at[b, 1], edge_out.at[1, pl.ds(0, CEB)],
                              osems[b]).wait()

    def do_chunk(c, b):
        wait_in(b)

        @pl.when(c >= 2)
        def _():
            wait_out(b)

        ew0 = [plsc.load_gather(attrW_v, [_splat(8 + k)]) for k in range(8)]
        ew1 = [plsc.load_gather(attrW_v, [_splat(16 + k)]) for k in range(8)]

        @plsc.parallel_loop(0, CE // 16, unroll=8)
        def egroup(g):
            teb = g // 8
            off = (g % 8) * 16
            rel16 = rel_v[b, pl.ds(g * 16, 16)]
            a0 = attrs_v[b, pl.ds(teb * 256 + off, 16)]
            a1 = attrs_v[b, pl.ds(teb * 256 + 128 + off, 16)]
            for col in range(EDGE_D):
                v = plsc.load_gather(etabT_v, [_splat(col * 5) + rel16])
                if col >= 8:
                    v = v + a0 * ew0[col - 8] + a1 * ew1[col - 8]
                eout_v[b, col // 8, teb, col % 8, pl.ds(off, 16)] = v

        tb = tb_base + c * CEB
        pltpu.async_copy(eout_v.at[b, 0], edge_out.at[0, pl.ds(tb, CEB)],
                         osems[b])
        pltpu.async_copy(eout_v.at[b, 1], edge_out.at[1, pl.ds(tb, CEB)],
                         osems[b])
        start_in(jnp.minimum(c + 2, nch - 1), b)

    start_in(0, 0)
    start_in(jnp.minimum(1, nch - 1), 1)

    def chunk_body(c, carry):
        @pl.when(c % 2 == 0)
        def _():
            do_chunk(c, 0)

        @pl.when(c % 2 == 1)
        def _():
            do_chunk(c, 1)

        return carry

    lax.fori_loop(0, nch, chunk_body, 0)
    wait_in(0)
    wait_in(1)
    wait_out(0)
    wait_out(1)

    nbase = wid * NPT
    ncount = jnp.minimum(NPT, N_NODES - nbase)

    def node_at(nb):
        pltpu.sync_copy(tidx_hbm.at[pl.ds(nb, CN)], tidx_v)
        pltpu.sync_copy(p0_hbm.at[pl.ds(nb, CN)], p0_v)
        pltpu.sync_copy(p1_hbm.at[pl.ds(nb, CN)], p1_v)

        @plsc.parallel_loop(0, CN, unroll=4)
        def nrow(n):
            nsp = _splat(0) + n
            t = plsc.load_gather(tidx_v, [nsp])
            q0 = plsc.load_gather(p0_v, [nsp])
            q1 = plsc.load_gather(p1_v, [nsp])
            rowbase = t * NODE_D
            for j in range(4):
                v = plsc.load_gather(ntab_v, [rowbase + (j * 16) + iota])
                nout_v[n // 8, n % 8, pl.ds(j * 16, 16)] = v
            for j in range(4):
                v = nbias[j] + q0 * nw0[j] + q1 * nw1[j]
                nout_v[n // 8, n % 8, pl.ds(64 + j * 16, 16)] = v

        pltpu.sync_copy(nout_v, node_out.at[pl.ds(nb // 8, CN // 8)])

    n_full = ncount // CN

    def node_chunk(c, carry):
        node_at(nbase + c * CN)
        return carry

    lax.fori_loop(0, n_full, node_chunk, 0)

    rem = ncount - n_full * CN

    @pl.when(rem > 0)
    def _():
        node_at(nbase + ncount - CN)


def kernel(type_indices, positions, relation_indices, edge_attrs,
           node_type_table, edge_type_table, pos_W, pos_b, attr_W, attr_b):
    ntab = jnp.concatenate(
        [node_type_table, jnp.broadcast_to(pos_b, (5, NODE_D // 2))],
        axis=1).reshape(-1)
    nwf = jnp.concatenate(
        [jnp.zeros((2, NODE_D // 2), jnp.float32), pos_W], axis=1).reshape(-1)
    etabT = jnp.concatenate(
        [edge_type_table.T,
         jnp.broadcast_to(attr_b[:, None], (EDGE_D // 2, 5))],
        axis=0).reshape(-1)
    etabT = jnp.pad(etabT, (0, 128 - etabT.shape[0]))
    attrW_pad = jnp.pad(attr_W.reshape(-1), (8, 120 - 2 * (EDGE_D // 2)))

    af = edge_attrs.reshape(NBLK, 128, 2).transpose(0, 2, 1).reshape(-1)
    p0 = positions[:, 0]
    p1 = positions[:, 1]

    mesh = plsc.VectorSubcoreMesh(core_axis_name="c", subcore_axis_name="s")
    fn = pl.kernel(
        _body,
        mesh=mesh,
        compiler_params=pltpu.CompilerParams(needs_layout_passes=False),
        out_type=[
            jax.ShapeDtypeStruct((N_NODES // 8, 8, NODE_D), jnp.float32),
            jax.ShapeDtypeStruct((2, NBLK, 8, 128), jnp.float32),
        ],
        scratch_types=[
            pltpu.VMEM((5 * NODE_D,), jnp.float32),
            pltpu.VMEM((2 * NODE_D,), jnp.float32),
            pltpu.VMEM((128,), jnp.float32),
            pltpu.VMEM((128,), jnp.float32),
            pltpu.VMEM((CN,), jnp.int32),
            pltpu.VMEM((CN,), jnp.float32),
            pltpu.VMEM((CN,), jnp.float32),
            pltpu.VMEM((CN // 8, 8, NODE_D), jnp.float32),
            pltpu.VMEM((2, CE), jnp.int32),
            pltpu.VMEM((2, 2 * CE), jnp.float32),
            pltpu.VMEM((2, 2, CEB, 8, 128), jnp.float32),
            pltpu.SemaphoreType.DMA,
            pltpu.SemaphoreType.DMA,
            pltpu.SemaphoreType.DMA,
            pltpu.SemaphoreType.DMA,
        ],
    )
    node3, edge4 = fn(
        type_indices, p0, p1, relation_indices, af,
        ntab, nwf, etabT, attrW_pad)
    node_features = node3.reshape(N_NODES, NODE_D)
    edge_attr = edge4.transpose(1, 3, 0, 2).reshape(N_EDGES, EDGE_D)
    return (node_features, edge_attr)
